# baseline (speedup 1.0000x reference)
import functools

import jax
import jax.numpy as jnp
from jax import lax
from jax.experimental import pallas as pl
from jax.experimental.pallas import tpu as pltpu
from jax.experimental.pallas import tpu_sc as plsc

VOCAB = 1000000
DIM = 32
FIELDS = 26
BATCH = 4096

_INFO = plsc.get_sparse_core_info()
_NC, _NS = _INFO.num_cores, _INFO.num_subcores
_NW = _NC * _NS
_TOTAL = FIELDS * BATCH
_PER_W = _TOTAL // _NW
_CHUNK = 128
_NCHUNK = _PER_W // _CHUNK


@functools.partial(
    pl.kernel,
    mesh=plsc.VectorSubcoreMesh(core_axis_name="c", subcore_axis_name="s"),
    out_type=jax.ShapeDtypeStruct((_TOTAL, DIM), jnp.float32),
    compiler_params=pltpu.CompilerParams(use_tc_tiling_on_sc=False),
    scratch_types=[
        pltpu.VMEM((_NCHUNK, _CHUNK), jnp.int32),
        pltpu.VMEM((_PER_W, DIM), jnp.float32),
        pltpu.SemaphoreType.DMA,
    ],
)
def _gather_kernel(idx_hbm, table_hbm, out_hbm, idx_v, rows_v, sem):
    wid = lax.axis_index("s") * _NC + lax.axis_index("c")
    pltpu.sync_copy(idx_hbm.at[wid], idx_v)
    copies = []
    for j in range(_NCHUNK):
        copies.append(
            pltpu.async_copy(
                table_hbm.at[idx_v.at[j]],
                rows_v.at[pl.ds(j * _CHUNK, _CHUNK)],
                sem,
            )
        )
    for cp in copies:
        cp.wait()
    pltpu.sync_copy(rows_v, out_hbm.at[pl.ds(wid * _PER_W, _PER_W)])


def kernel(indices, embedding_table):
    idx2 = indices.T.reshape(_NW, _NCHUNK, _CHUNK)
    out = _gather_kernel(idx2, embedding_table)
    return out.reshape(BATCH, FIELDS * DIM)


# speedup vs baseline: 1.7756x; 1.7756x over previous
import functools

import jax
import jax.numpy as jnp
from jax import lax
from jax.experimental import pallas as pl
from jax.experimental.pallas import tpu as pltpu
from jax.experimental.pallas import tpu_sc as plsc

VOCAB = 1000000
DIM = 32
FIELDS = 26
BATCH = 4096

_INFO = plsc.get_sparse_core_info()
_NC, _NS = _INFO.num_cores, _INFO.num_subcores
_NW = _NC * _NS
_TOTAL = FIELDS * BATCH
_PER_W = _TOTAL // _NW
_CHUNK = 128
_NCHUNK = _PER_W // _CHUNK


@functools.partial(
    pl.kernel,
    mesh=plsc.VectorSubcoreMesh(core_axis_name="c", subcore_axis_name="s"),
    out_type=jax.ShapeDtypeStruct((_TOTAL, DIM), jnp.float32),
    compiler_params=pltpu.CompilerParams(use_tc_tiling_on_sc=False),
    scratch_types=[
        pltpu.VMEM((_NCHUNK, _CHUNK), jnp.int32),
        pltpu.VMEM((_PER_W, DIM), jnp.float32),
        pltpu.SemaphoreType.DMA,
    ],
)
def _gather_kernel(idx_hbm, table_hbm, out_hbm, idx_v, rows_v, sem):
    wid = lax.axis_index("s") * _NC + lax.axis_index("c")
    pltpu.sync_copy(idx_hbm.at[wid], idx_v)
    copies = []
    for j in range(_NCHUNK):
        copies.append(
            pltpu.async_copy(
                table_hbm.at[idx_v.at[j]],
                rows_v.at[pl.ds(j * _CHUNK, _CHUNK)],
                sem,
            )
        )
    for cp in copies:
        cp.wait()
    pltpu.sync_copy(rows_v, out_hbm.at[pl.ds(wid * _PER_W, _PER_W)])


_VBLK = 8192
_GRP = _VBLK // 4
_TGRID = -(-VOCAB // _VBLK)
_VPAD = _TGRID * _VBLK


def _transpose_body(x_ref, o_ref):
    x = x_ref[...]
    o_ref[...] = jnp.concatenate(
        [x[:, g * _GRP:(g + 1) * _GRP].T for g in range(4)], axis=1
    )


_transpose_call = pl.pallas_call(
    _transpose_body,
    grid=(_TGRID,),
    in_specs=[pl.BlockSpec((DIM, _VBLK), lambda g: (0, g))],
    out_specs=pl.BlockSpec((_VBLK * DIM // 128, 128), lambda g: (g, 0)),
    out_shape=jax.ShapeDtypeStruct((_VPAD * DIM // 128, 128), jnp.float32),
)


def kernel(indices, embedding_table):
    vt = indices.T
    u = ((vt >> 13) << 13) | ((vt & 2047) << 2) | ((vt >> 11) & 3)
    idx2 = u.reshape(_NW, _NCHUNK, _CHUNK)
    t = _transpose_call(embedding_table.T)
    t = t.reshape(_VPAD, DIM)
    out = _gather_kernel(idx2, t)
    return out.reshape(BATCH, FIELDS * DIM)


# speedup vs baseline: 2.7547x; 1.5514x over previous
import functools

import jax
import jax.numpy as jnp
from jax import lax
from jax.experimental import pallas as pl
from jax.experimental.pallas import tpu as pltpu
from jax.experimental.pallas import tpu_sc as plsc

VOCAB = 1000000
DIM = 32
FIELDS = 26
BATCH = 4096

_INFO = plsc.get_sparse_core_info()
_NC, _NS = _INFO.num_cores, _INFO.num_subcores
_NW = _NC * _NS
_TOTAL = FIELDS * BATCH
_PER_W = _TOTAL // _NW
_CHUNK = 128
_NCHUNK = _PER_W // _CHUNK


@functools.partial(
    pl.kernel,
    mesh=plsc.VectorSubcoreMesh(core_axis_name="c", subcore_axis_name="s"),
    out_type=jax.ShapeDtypeStruct((_TOTAL, DIM), jnp.float32),
    compiler_params=pltpu.CompilerParams(use_tc_tiling_on_sc=False),
    scratch_types=[
        pltpu.VMEM((_NCHUNK, _CHUNK), jnp.int32),
        pltpu.VMEM((_PER_W, DIM), jnp.float32),
        pltpu.SemaphoreType.DMA,
    ],
)
def _gather_kernel(idx_hbm, table_hbm, out_hbm, idx_v, rows_v, sem):
    wid = lax.axis_index("s") * _NC + lax.axis_index("c")
    pltpu.sync_copy(idx_hbm.at[wid], idx_v)
    copies = []
    for j in range(_NCHUNK):
        copies.append(
            pltpu.async_copy(
                table_hbm.at[idx_v.at[j]],
                rows_v.at[pl.ds(j * _CHUNK, _CHUNK)],
                sem,
            )
        )
    for cp in copies:
        cp.wait()
    pltpu.sync_copy(rows_v, out_hbm.at[pl.ds(wid * _PER_W, _PER_W)])


_VBLK = 8192
_GRP = _VBLK // 4
_TGRID = -(-VOCAB // _VBLK)
_VPAD = _TGRID * _VBLK


def _transpose_body(x_ref, o_ref):
    x = x_ref[...]
    stacked = jnp.concatenate(
        [x[:, g * _GRP:(g + 1) * _GRP] for g in range(4)], axis=0
    )
    o_ref[...] = stacked.T


_transpose_call = pl.pallas_call(
    _transpose_body,
    grid=(_TGRID,),
    in_specs=[pl.BlockSpec((DIM, _VBLK), lambda g: (0, g))],
    out_specs=pl.BlockSpec((_VBLK * DIM // 128, 128), lambda g: (g, 0)),
    out_shape=jax.ShapeDtypeStruct((_VPAD * DIM // 128, 128), jnp.float32),
)


def kernel(indices, embedding_table):
    vt = indices.T
    u = ((vt >> 13) << 13) | ((vt & 2047) << 2) | ((vt >> 11) & 3)
    idx2 = u.reshape(_NW, _NCHUNK, _CHUNK)
    t = _transpose_call(embedding_table.T)
    t = t.reshape(_VPAD, DIM)
    out = _gather_kernel(idx2, t)
    return out.reshape(BATCH, FIELDS * DIM)


# speedup vs baseline: 3.4153x; 1.2398x over previous
import functools

import jax
import jax.numpy as jnp
from jax import lax
from jax.experimental import pallas as pl
from jax.experimental.pallas import tpu as pltpu
from jax.experimental.pallas import tpu_sc as plsc

VOCAB = 1000000
DIM = 32
FIELDS = 26
BATCH = 4096

_INFO = plsc.get_sparse_core_info()
_NC, _NS = _INFO.num_cores, _INFO.num_subcores
_NW = _NC * _NS
_TOTAL = FIELDS * BATCH
_PER_W = _TOTAL // _NW
_CHUNK = 128
_NCHUNK = _PER_W // _CHUNK


@functools.partial(
    pl.kernel,
    mesh=plsc.VectorSubcoreMesh(core_axis_name="c", subcore_axis_name="s"),
    out_type=jax.ShapeDtypeStruct((_TOTAL, DIM), jnp.float32),
    compiler_params=pltpu.CompilerParams(use_tc_tiling_on_sc=False),
    scratch_types=[
        pltpu.VMEM((_NCHUNK, _CHUNK), jnp.int32),
        pltpu.VMEM((_PER_W, DIM), jnp.float32),
        pltpu.SemaphoreType.DMA,
    ],
)
def _gather_kernel(idx_hbm, table_hbm, out_hbm, idx_v, rows_v, sem):
    wid = lax.axis_index("s") * _NC + lax.axis_index("c")
    pltpu.sync_copy(idx_hbm.at[wid], idx_v)
    copies = []
    for j in range(_NCHUNK):
        copies.append(
            pltpu.async_copy(
                table_hbm.at[idx_v.at[j]],
                rows_v.at[pl.ds(j * _CHUNK, _CHUNK)],
                sem,
            )
        )
    for cp in copies:
        cp.wait()
    pltpu.sync_copy(rows_v, out_hbm.at[pl.ds(wid * _PER_W, _PER_W)])


_VBLK = 16384
_GRP = _VBLK // 4
_TGRID = -(-VOCAB // _VBLK)
_VPAD = _TGRID * _VBLK
_SB = _VBLK.bit_length() - 1
_SG = _GRP.bit_length() - 1


def _transpose_body(x_ref, o_ref):
    x = x_ref[...]
    stacked = jnp.concatenate(
        [x[:, g * _GRP:(g + 1) * _GRP] for g in range(4)], axis=0
    )
    o_ref[...] = stacked.T


_transpose_call = pl.pallas_call(
    _transpose_body,
    grid=(_TGRID,),
    in_specs=[pl.BlockSpec((DIM, _VBLK), lambda g: (0, g))],
    out_specs=pl.BlockSpec((_VBLK * DIM // 128, 128), lambda g: (g, 0)),
    out_shape=jax.ShapeDtypeStruct((_VPAD * DIM // 128, 128), jnp.float32),
)


def kernel(indices, embedding_table):
    vt = indices.T
    u = ((vt >> _SB) << _SB) | ((vt & (_GRP - 1)) << 2) | ((vt >> _SG) & 3)
    idx2 = u.reshape(_NW, _NCHUNK, _CHUNK)
    t = _transpose_call(embedding_table.T)
    t = t.reshape(_VPAD, DIM)
    out = _gather_kernel(idx2, t)
    return out.reshape(BATCH, FIELDS * DIM)


# speedup vs baseline: 3.7577x; 1.1002x over previous
import functools

import jax
import jax.numpy as jnp
from jax import lax
from jax.experimental import pallas as pl
from jax.experimental.pallas import tpu as pltpu
from jax.experimental.pallas import tpu_sc as plsc

VOCAB = 1000000
DIM = 32
FIELDS = 26
BATCH = 4096

_INFO = plsc.get_sparse_core_info()
_NC, _NS = _INFO.num_cores, _INFO.num_subcores
_NW = _NC * _NS
_TOTAL = FIELDS * BATCH
_PER_W = _TOTAL // _NW
_CHUNK = 128
_NCHUNK = _PER_W // _CHUNK


@functools.partial(
    pl.kernel,
    mesh=plsc.VectorSubcoreMesh(core_axis_name="c", subcore_axis_name="s"),
    out_type=jax.ShapeDtypeStruct((_TOTAL, DIM), jnp.float32),
    compiler_params=pltpu.CompilerParams(use_tc_tiling_on_sc=False),
    scratch_types=[
        pltpu.VMEM((_NCHUNK, _CHUNK), jnp.int32),
        pltpu.VMEM((_PER_W, DIM), jnp.float32),
        pltpu.SemaphoreType.DMA,
    ],
)
def _gather_kernel(idx_hbm, table_hbm, out_hbm, idx_v, rows_v, sem):
    wid = lax.axis_index("s") * _NC + lax.axis_index("c")
    pltpu.sync_copy(idx_hbm.at[wid], idx_v)
    copies = []
    for j in range(_NCHUNK):
        copies.append(
            pltpu.async_copy(
                table_hbm.at[idx_v.at[j]],
                rows_v.at[pl.ds(j * _CHUNK, _CHUNK)],
                sem,
            )
        )
    for cp in copies:
        cp.wait()
    pltpu.sync_copy(rows_v, out_hbm.at[pl.ds(wid * _PER_W, _PER_W)])


_VBLK = 32768
_GRP = _VBLK // 4
_TGRID = -(-VOCAB // _VBLK)
_VPAD = _TGRID * _VBLK
_SB = _VBLK.bit_length() - 1
_SG = _GRP.bit_length() - 1


def _transpose_body(x_ref, o_ref):
    x = x_ref[...]
    stacked = jnp.concatenate(
        [x[:, g * _GRP:(g + 1) * _GRP] for g in range(4)], axis=0
    )
    o_ref[...] = stacked.T


_transpose_call = pl.pallas_call(
    _transpose_body,
    grid=(_TGRID,),
    in_specs=[pl.BlockSpec((DIM, _VBLK), lambda g: (0, g))],
    out_specs=pl.BlockSpec((_VBLK * DIM // 128, 128), lambda g: (g, 0)),
    out_shape=jax.ShapeDtypeStruct((_VPAD * DIM // 128, 128), jnp.float32),
)


def kernel(indices, embedding_table):
    vt = indices.T
    u = ((vt >> _SB) << _SB) | ((vt & (_GRP - 1)) << 2) | ((vt >> _SG) & 3)
    idx2 = u.reshape(_NW, _NCHUNK, _CHUNK)
    t = _transpose_call(embedding_table.T)
    t = t.reshape(_VPAD, DIM)
    out = _gather_kernel(idx2, t)
    return out.reshape(BATCH, FIELDS * DIM)


# speedup vs baseline: 3.8001x; 1.0113x over previous
import functools

import jax
import jax.numpy as jnp
from jax import lax
from jax.experimental import pallas as pl
from jax.experimental.pallas import tpu as pltpu
from jax.experimental.pallas import tpu_sc as plsc

VOCAB = 1000000
DIM = 32
FIELDS = 26
BATCH = 4096

_INFO = plsc.get_sparse_core_info()
_NC, _NS = _INFO.num_cores, _INFO.num_subcores
_NW = _NC * _NS
_TOTAL = FIELDS * BATCH
_PER_W = _TOTAL // _NW
_CHUNK = 128
_NCHUNK = _PER_W // _CHUNK


@functools.partial(
    pl.kernel,
    mesh=plsc.VectorSubcoreMesh(core_axis_name="c", subcore_axis_name="s"),
    out_type=jax.ShapeDtypeStruct((_TOTAL, DIM), jnp.float32),
    compiler_params=pltpu.CompilerParams(use_tc_tiling_on_sc=False),
    scratch_types=[
        pltpu.VMEM((_NCHUNK, _CHUNK), jnp.int32),
        pltpu.VMEM((_PER_W, DIM), jnp.float32),
        pltpu.SemaphoreType.DMA,
    ],
)
def _gather_kernel(idx_hbm, table_hbm, out_hbm, idx_v, rows_v, sem):
    wid = lax.axis_index("s") * _NC + lax.axis_index("c")
    pltpu.sync_copy(idx_hbm.at[wid], idx_v)
    copies = []
    for j in range(_NCHUNK):
        copies.append(
            pltpu.async_copy(
                table_hbm.at[idx_v.at[j]],
                rows_v.at[pl.ds(j * _CHUNK, _CHUNK)],
                sem,
            )
        )
    for cp in copies:
        cp.wait()
    pltpu.sync_copy(rows_v, out_hbm.at[pl.ds(wid * _PER_W, _PER_W)])


_VBLK = 65536
_GRP = _VBLK // 4
_TGRID = -(-VOCAB // _VBLK)
_VPAD = _TGRID * _VBLK
_SB = _VBLK.bit_length() - 1
_SG = _GRP.bit_length() - 1


def _transpose_body(x_ref, o_ref):
    x = x_ref[...]
    stacked = jnp.concatenate(
        [x[:, g * _GRP:(g + 1) * _GRP] for g in range(4)], axis=0
    )
    o_ref[...] = stacked.T


_transpose_call = pl.pallas_call(
    _transpose_body,
    grid=(_TGRID,),
    in_specs=[pl.BlockSpec((DIM, _VBLK), lambda g: (0, g))],
    out_specs=pl.BlockSpec((_VBLK * DIM // 128, 128), lambda g: (g, 0)),
    out_shape=jax.ShapeDtypeStruct((_VPAD * DIM // 128, 128), jnp.float32),
)


def kernel(indices, embedding_table):
    vt = indices.T
    u = ((vt >> _SB) << _SB) | ((vt & (_GRP - 1)) << 2) | ((vt >> _SG) & 3)
    idx2 = u.reshape(_NW, _NCHUNK, _CHUNK)
    t = _transpose_call(embedding_table.T)
    t = t.reshape(_VPAD, DIM)
    out = _gather_kernel(idx2, t)
    return out.reshape(BATCH, FIELDS * DIM)
